# baseline (speedup 1.0000x reference)
import functools

import jax
import jax.numpy as jnp
from jax import lax
from jax.experimental import pallas as pl
from jax.experimental.pallas import tpu as pltpu
from jax.experimental.pallas import tpu_sc as plsc

T = 2048
BEAM = 8
SRC = 2048
NBEST = 4
NS = T - 1
ROWS = NBEST * T

_MESH = dict(core_axis_name="c", subcore_axis_name="s", num_cores=2,
             num_subcores=16)


def _stage1_body(tokens_hbm, scores_hbm, prev_hbm,
                 ord_hbm, sc_hbm, tok_hbm, tls_hbm, rows_hbm,
                 tokens_v, scores_v, prev_v,
                 rows_v, tokbuf_v, tlsbuf_v, misci_v, miscf_v):
    cid = lax.axis_index("c")
    sid = lax.axis_index("s")

    @pl.when(jnp.logical_and(cid == 0, sid == 0))
    def _():
        pltpu.sync_copy(tokens_hbm, tokens_v)
        pltpu.sync_copy(scores_hbm, scores_v)
        pltpu.sync_copy(prev_hbm, prev_v)

        lane = lax.broadcasted_iota(jnp.int32, (16,), 0)
        mask4 = lane < NBEST

        sc_last = scores_v[pl.ds(T * BEAM - 16, 16)]
        neg = jnp.float32(-jnp.inf)
        cand = jnp.where(lane >= 8, sc_last, neg)
        b = jnp.zeros((16,), jnp.int32)
        for i in range(NBEST):
            m = cand
            for sh in (1, 2, 4, 8):
                rot = m.at[jnp.bitwise_and(lane + sh, 15)].get(
                    mode="promise_in_bounds")
                m = jnp.maximum(m, rot)
            j = plsc.all_reduce_ffs(cand == m)
            b = jnp.where(lane == i, j - 8, b)
            cand = jnp.where(lane == j, neg, cand)

        misci_v[...] = jnp.where(mask4, b, 0)
        sc4 = plsc.load_gather(scores_v, [NS * BEAM + b])
        miscf_v[...] = jnp.where(mask4, sc4, jnp.float32(0.0))
        pltpu.sync_copy(misci_v, ord_hbm)
        pltpu.sync_copy(miscf_v, sc_hbm)

        base4 = lane * T
        plsc.store_scatter(rows_v, [base4 + NS], jnp.zeros((16,), jnp.int32),
                           mask=mask4)

        def bt_step(t, bcur):
            idx = t * BEAM + bcur
            plsc.store_scatter(rows_v, [base4 + (t - 1)], idx, mask=mask4)
            return plsc.load_gather(prev_v, [idx])

        def bt_body(k, bcur):
            t0 = NS - k * 8
            for u in range(8):
                bcur = bt_step(t0 - u, bcur)
            return bcur

        b_tail = lax.fori_loop(0, NS // 8, bt_body, b)
        for t in range(NS - (NS // 8) * 8, 0, -1):
            b_tail = bt_step(t, b_tail)

        rotm1 = jnp.bitwise_and(lane + 15, 16 - 1)
        for i in range(NBEST):
            def blk_body(v, carry, i=i):
                off = i * T + v * 16
                ivec = rows_v[pl.ds(off, 16)]
                tokbuf_v[pl.ds(off, 16)] = plsc.load_gather(tokens_v, [ivec])
                sc = plsc.load_gather(scores_v, [ivec])
                srot = sc.at[rotm1].get(mode="promise_in_bounds")
                prev_sc = jnp.where(lane == 0, carry, srot)
                tlsbuf_v[pl.ds(off, 16)] = sc - prev_sc
                return sc[15]

            lax.fori_loop(0, T // 16, blk_body, jnp.float32(0.0))

        pltpu.sync_copy(tokbuf_v, tok_hbm)
        pltpu.sync_copy(tlsbuf_v, tls_hbm)
        pltpu.sync_copy(rows_v, rows_hbm)


_stage1 = functools.partial(
    pl.kernel,
    out_type=[
        jax.ShapeDtypeStruct((16,), jnp.int32),
        jax.ShapeDtypeStruct((16,), jnp.float32),
        jax.ShapeDtypeStruct((ROWS,), jnp.int32),
        jax.ShapeDtypeStruct((ROWS,), jnp.float32),
        jax.ShapeDtypeStruct((ROWS,), jnp.int32),
    ],
    mesh=plsc.VectorSubcoreMesh(**_MESH),
    compiler_params=pltpu.CompilerParams(needs_layout_passes=False),
    scratch_types=[
        pltpu.VMEM((T * BEAM,), jnp.int32),
        pltpu.VMEM((T * BEAM,), jnp.float32),
        pltpu.VMEM((T * BEAM,), jnp.int32),
        pltpu.VMEM((ROWS,), jnp.int32),
        pltpu.VMEM((ROWS,), jnp.int32),
        pltpu.VMEM((ROWS,), jnp.float32),
        pltpu.VMEM((16,), jnp.int32),
        pltpu.VMEM((16,), jnp.float32),
    ],
)(_stage1_body)


_PER_W = T // 8
_TB = 128
_NCB = _PER_W // _TB
_NSB = SRC // _TB
_NBLK = _NCB * _NSB


def _stage2t_body(tw_hbm, rows_hbm, o0, o1, o2, o3, tail_hbm,
                  idx_v, g0, g1, t0, t1, gs0, gs1, ws0, ws1):
    cid = lax.axis_index("c")
    sid = lax.axis_index("s")
    wid = sid * 2 + cid
    base = wid * _PER_W
    pltpu.sync_copy(rows_hbm.at[pl.ds(base, _PER_W)], idx_v)
    hyp = wid // 8
    r = wid % 8
    ts = r * _PER_W

    lane = lax.broadcasted_iota(jnp.int32, (16,), 0)
    zeros = jnp.zeros((16,), jnp.int32)
    gbufs = (g0, g1)
    gsems = (gs0, gs1)
    tbufs = (t0, t1)
    wsems = (ws0, ws1)
    outs = (o0, o1, o2, o3)

    def fire(k, par):
        cb, sb = k // _NSB, k % _NSB
        pltpu.async_copy(
            tw_hbm.at[idx_v.at[pl.ds(cb * _TB, _TB)],
                      pl.ds(sb * _TB, _TB)],
            gbufs[par], gsems[par])

    def drain_gather(par):
        pltpu.make_async_copy(
            tw_hbm.at[pl.ds(0, _TB), pl.ds(0, _TB)],
            gbufs[par], gsems[par]).wait()

    def transpose_blk(gbuf, tbuf):
        def body(s, _):
            for jv in range(_TB // 16):
                vec = plsc.load_gather(gbuf, [jv * 16 + lane, zeros + s])
                plsc.store_scatter(tbuf, [zeros + s, jv * 16 + lane], vec)
            return 0

        lax.fori_loop(0, _TB, body, 0)

    def drain_write(p2):
        pltpu.make_async_copy(
            tbufs[p2],
            outs[0].at[pl.ds(0, _TB), pl.ds(0, _TB)],
            wsems[p2]).wait()

    fire(0, 0)

    def pair_body(kk, _):
        for par in (0, 1):
            k = 2 * kk + par
            cb, sb = k // _NSB, k % _NSB

            @pl.when(k + 1 < _NBLK)
            def _(par=par, k=k):
                fire(k + 1, 1 - par)

            drain_gather(par)

            @pl.when(k >= 2)
            def _(par=par):
                drain_write(par)

            transpose_blk(gbufs[par], tbufs[par])
            to_tail = jnp.logical_and(cb == _NCB - 1, r == 8 - 1)
            for hy in range(NBEST):
                @pl.when(jnp.logical_and(hyp == hy, to_tail))
                def _(hy=hy, par=par, sb=sb):
                    pltpu.async_copy(
                        tbufs[par],
                        tail_hbm.at[hy, pl.ds(sb * _TB, _TB), pl.ds(0, _TB)],
                        wsems[par])

                @pl.when(jnp.logical_and(hyp == hy,
                                         jnp.logical_not(to_tail)))
                def _(hy=hy, par=par, sb=sb, cb=cb):
                    pltpu.async_copy(
                        tbufs[par],
                        outs[hy].at[pl.ds(sb * _TB, _TB),
                                    pl.ds(ts + cb * _TB, _TB)],
                        wsems[par])
        return 0

    lax.fori_loop(0, _NBLK // 2, pair_body, 0)
    drain_write(0)
    drain_write(1)


_stage2t = functools.partial(
    pl.kernel,
    out_type=[jax.ShapeDtypeStruct((SRC, NS), jnp.float32)
              for _ in range(NBEST)] +
             [jax.ShapeDtypeStruct((NBEST, SRC, _TB), jnp.float32)],
    mesh=plsc.VectorSubcoreMesh(**_MESH),
    compiler_params=pltpu.CompilerParams(needs_layout_passes=False),
    scratch_types=[
        pltpu.VMEM((_PER_W,), jnp.int32),
        pltpu.VMEM((_TB, _TB), jnp.float32),
        pltpu.VMEM((_TB, _TB), jnp.float32),
        pltpu.VMEM((_TB, _TB), jnp.float32),
        pltpu.VMEM((_TB, _TB), jnp.float32),
        pltpu.SemaphoreType.DMA,
        pltpu.SemaphoreType.DMA,
        pltpu.SemaphoreType.DMA,
        pltpu.SemaphoreType.DMA,
    ],
)(_stage2t_body)


def _fix_body(tail_ref, a0, a1, a2, a3, o0, o1, o2, o3):
    del a0, a1, a2, a3
    for hy, o in enumerate((o0, o1, o2, o3)):
        o[...] = tail_ref[hy]


def _tail_fix(tail, baw):
    return pl.pallas_call(
        _fix_body,
        grid=(1,),
        in_specs=[pl.BlockSpec((NBEST, SRC, _TB), lambda i: (0, 0, 0))] +
                 [pl.BlockSpec((8, 128), lambda i: (0, 0))
                  for _ in range(NBEST)],
        out_specs=[pl.BlockSpec((SRC, _TB), lambda i: (0, NS // _TB))
                   for _ in range(NBEST)],
        out_shape=[jax.ShapeDtypeStruct((SRC, NS), jnp.float32)
                   for _ in range(NBEST)],
        input_output_aliases={1: 0, 2: 1, 3: 2, 4: 3},
    )(tail, *baw)


def kernel(beam_tokens, beam_scores, token_weights, beam_prev_indices,
           num_steps):
    tokens_flat = beam_tokens.reshape(-1)
    scores_flat = beam_scores.reshape(-1)
    prev_flat = beam_prev_indices.reshape(-1)
    tw_flat = token_weights.reshape(T * BEAM, SRC)

    ord16, sc16, tokf, tlsf, rows = _stage1(tokens_flat, scores_flat,
                                            prev_flat)
    *baw_sc, tail = _stage2t(tw_flat, rows)
    baw = _tail_fix(tail, baw_sc)

    ns_t = jnp.asarray(num_steps, jnp.int32)
    ns_f = ns_t.astype(jnp.float32)
    tok4 = tokf.reshape(NBEST, T)
    tls4 = tlsf.reshape(NBEST, T)
    outs = []
    for i in range(NBEST):
        outs.extend([
            tok4[i, :NS],
            sc16[i] / ns_f,
            tls4[i, :NS],
            baw[i],
            jnp.stack([ns_t, ord16[i]]).astype(jnp.int32),
        ])
    return tuple(outs)


# speedup vs baseline: 3.1303x; 3.1303x over previous
import functools

import jax
import jax.numpy as jnp
from jax import lax
from jax.experimental import pallas as pl
from jax.experimental.pallas import tpu as pltpu
from jax.experimental.pallas import tpu_sc as plsc

T = 2048
BEAM = 8
SRC = 2048
NBEST = 4
NS = T - 1
ROWS = NBEST * T

_MESH = dict(core_axis_name="c", subcore_axis_name="s", num_cores=2,
             num_subcores=16)


def _stage1_body(tokens_hbm, scores_hbm, prev_hbm,
                 ord_hbm, sc_hbm, tok_hbm, tls_hbm, rows_hbm,
                 tokens_v, scores_v, prev_v,
                 rows_v, tokbuf_v, tlsbuf_v, misci_v, miscf_v):
    cid = lax.axis_index("c")
    sid = lax.axis_index("s")

    @pl.when(jnp.logical_and(cid == 0, sid == 0))
    def _():
        pltpu.sync_copy(tokens_hbm, tokens_v)
        pltpu.sync_copy(scores_hbm, scores_v)
        pltpu.sync_copy(prev_hbm, prev_v)

        lane = lax.broadcasted_iota(jnp.int32, (16,), 0)
        mask4 = lane < NBEST

        sc_last = scores_v[pl.ds(T * BEAM - 16, 16)]
        neg = jnp.float32(-jnp.inf)
        cand = jnp.where(lane >= 8, sc_last, neg)
        b = jnp.zeros((16,), jnp.int32)
        for i in range(NBEST):
            m = cand
            for sh in (1, 2, 4, 8):
                rot = m.at[jnp.bitwise_and(lane + sh, 15)].get(
                    mode="promise_in_bounds")
                m = jnp.maximum(m, rot)
            j = plsc.all_reduce_ffs(cand == m)
            b = jnp.where(lane == i, j - 8, b)
            cand = jnp.where(lane == j, neg, cand)

        misci_v[...] = jnp.where(mask4, b, 0)
        sc4 = plsc.load_gather(scores_v, [NS * BEAM + b])
        miscf_v[...] = jnp.where(mask4, sc4, jnp.float32(0.0))
        pltpu.sync_copy(misci_v, ord_hbm)
        pltpu.sync_copy(miscf_v, sc_hbm)

        base4 = lane * T
        plsc.store_scatter(rows_v, [base4 + NS], jnp.zeros((16,), jnp.int32),
                           mask=mask4)

        def bt_step(t, bcur):
            idx = t * BEAM + bcur
            plsc.store_scatter(rows_v, [base4 + (t - 1)], idx, mask=mask4)
            return plsc.load_gather(prev_v, [idx])

        def bt_body(k, bcur):
            t0 = NS - k * 8
            for u in range(8):
                bcur = bt_step(t0 - u, bcur)
            return bcur

        b_tail = lax.fori_loop(0, NS // 8, bt_body, b)
        for t in range(NS - (NS // 8) * 8, 0, -1):
            b_tail = bt_step(t, b_tail)

        rotm1 = jnp.bitwise_and(lane + 15, 16 - 1)
        for i in range(NBEST):
            def blk_body(v, carry, i=i):
                off = i * T + v * 16
                ivec = rows_v[pl.ds(off, 16)]
                tokbuf_v[pl.ds(off, 16)] = plsc.load_gather(tokens_v, [ivec])
                sc = plsc.load_gather(scores_v, [ivec])
                srot = sc.at[rotm1].get(mode="promise_in_bounds")
                prev_sc = jnp.where(lane == 0, carry, srot)
                tlsbuf_v[pl.ds(off, 16)] = sc - prev_sc
                return sc[15]

            lax.fori_loop(0, T // 16, blk_body, jnp.float32(0.0))

        pltpu.sync_copy(tokbuf_v, tok_hbm)
        pltpu.sync_copy(tlsbuf_v, tls_hbm)
        pltpu.sync_copy(rows_v, rows_hbm)


_stage1 = functools.partial(
    pl.kernel,
    out_type=[
        jax.ShapeDtypeStruct((16,), jnp.int32),
        jax.ShapeDtypeStruct((16,), jnp.float32),
        jax.ShapeDtypeStruct((ROWS,), jnp.int32),
        jax.ShapeDtypeStruct((ROWS,), jnp.float32),
        jax.ShapeDtypeStruct((ROWS,), jnp.int32),
    ],
    mesh=plsc.VectorSubcoreMesh(**_MESH),
    compiler_params=pltpu.CompilerParams(needs_layout_passes=False),
    scratch_types=[
        pltpu.VMEM((T * BEAM,), jnp.int32),
        pltpu.VMEM((T * BEAM,), jnp.float32),
        pltpu.VMEM((T * BEAM,), jnp.int32),
        pltpu.VMEM((ROWS,), jnp.int32),
        pltpu.VMEM((ROWS,), jnp.int32),
        pltpu.VMEM((ROWS,), jnp.float32),
        pltpu.VMEM((16,), jnp.int32),
        pltpu.VMEM((16,), jnp.float32),
    ],
)(_stage1_body)


_CHUNK = 16
_NH = 4
_NGROUP = NBEST // _NH
_GROWS = _NH * T
_PER_W = _GROWS // 32
_NCHUNK = _PER_W // _CHUNK


def _stage2_body(tw_hbm, rows_hbm, out_hbm, idx_v, buf0, buf1,
                 gsem0, gsem1, wsem0, wsem1):
    wid = lax.axis_index("s") * 2 + lax.axis_index("c")
    base = wid * _PER_W
    pltpu.sync_copy(rows_hbm.at[pl.ds(base, _PER_W)], idx_v)

    bufs = (buf0, buf1)
    gsems = (gsem0, gsem1)
    wsems = (wsem0, wsem1)

    def fire(c):
        return pltpu.async_copy(
            tw_hbm.at[idx_v.at[pl.ds(c * _CHUNK, _CHUNK)]],
            bufs[c & 1], gsems[c & 1])

    gcp = {0: fire(0)}
    wcp = {}
    for c in range(_NCHUNK):
        p = c & 1
        q = (c + 1) & 1
        if c + 1 < _NCHUNK:
            if c >= 1:
                wcp[q].wait()
            gcp[q] = fire(c + 1)
        gcp[p].wait()
        wcp[p] = pltpu.async_copy(
            bufs[p], out_hbm.at[pl.ds(base + c * _CHUNK, _CHUNK)], wsems[p])
    wcp[(_NCHUNK - 2) & 1].wait()
    wcp[(_NCHUNK - 1) & 1].wait()


_stage2 = functools.partial(
    pl.kernel,
    out_type=jax.ShapeDtypeStruct((_GROWS, SRC), jnp.float32),
    mesh=plsc.VectorSubcoreMesh(**_MESH),
    compiler_params=pltpu.CompilerParams(needs_layout_passes=False),
    scratch_types=[
        pltpu.VMEM((_PER_W,), jnp.int32),
        pltpu.VMEM((_CHUNK, SRC), jnp.float32),
        pltpu.VMEM((_CHUNK, SRC), jnp.float32),
        pltpu.SemaphoreType.DMA,
        pltpu.SemaphoreType.DMA,
        pltpu.SemaphoreType.DMA,
        pltpu.SemaphoreType.DMA,
    ],
)(_stage2_body)


_TT = 256


def _tr_body(*refs):
    xs, os = refs[:_NH], refs[_NH:]
    for x, o in zip(xs, os):
        o[...] = jnp.swapaxes(x[0], 0, 1)


def _stage3(compact):
    in_specs = [
        pl.BlockSpec((1, _TT, SRC), lambda tb, k=k: (k, tb, 0))
        for k in range(_NH)
    ]
    out_specs = [
        pl.BlockSpec((SRC, _TT), lambda tb: (0, tb)) for _ in range(_NH)
    ]
    out_shape = [
        jax.ShapeDtypeStruct((SRC, NS), jnp.float32) for _ in range(_NH)
    ]
    return pl.pallas_call(
        _tr_body,
        grid=(T // _TT,),
        in_specs=in_specs,
        out_specs=out_specs,
        out_shape=out_shape,
        compiler_params=pltpu.CompilerParams(
            vmem_limit_bytes=100 * 1024 * 1024),
    )(*([compact] * _NH))


def kernel(beam_tokens, beam_scores, token_weights, beam_prev_indices,
           num_steps):
    tokens_flat = beam_tokens.reshape(-1)
    scores_flat = beam_scores.reshape(-1)
    prev_flat = beam_prev_indices.reshape(-1)
    tw_flat = token_weights.reshape(T * BEAM, SRC)

    ord16, sc16, tokf, tlsf, rows = _stage1(tokens_flat, scores_flat,
                                            prev_flat)
    baw = []
    for g in range(_NGROUP):
        rows_g = lax.slice(rows, (g * _GROWS,), ((g + 1) * _GROWS,))
        compact = _stage2(tw_flat, rows_g)
        baw.extend(_stage3(compact.reshape(_NH, T, SRC)))

    ns_t = jnp.asarray(num_steps, jnp.int32)
    ns_f = ns_t.astype(jnp.float32)
    tok4 = tokf.reshape(NBEST, T)
    tls4 = tlsf.reshape(NBEST, T)
    outs = []
    for i in range(NBEST):
        outs.extend([
            tok4[i, :NS],
            sc16[i] / ns_f,
            tls4[i, :NS],
            baw[i],
            jnp.stack([ns_t, ord16[i]]).astype(jnp.int32),
        ])
    return tuple(outs)


# speedup vs baseline: 3.1700x; 1.0127x over previous
import functools

import jax
import jax.numpy as jnp
from jax import lax
from jax.experimental import pallas as pl
from jax.experimental.pallas import tpu as pltpu
from jax.experimental.pallas import tpu_sc as plsc

T = 2048
BEAM = 8
SRC = 2048
NBEST = 4
NS = T - 1
ROWS = NBEST * T

_MESH = dict(core_axis_name="c", subcore_axis_name="s", num_cores=2,
             num_subcores=16)


def _stage1_body(tokens_hbm, scores_hbm, prev_hbm,
                 ord_hbm, sc_hbm, tok_hbm, tls_hbm, rows_hbm,
                 tokens_v, scores_v, prev_v,
                 rows_v, tokbuf_v, tlsbuf_v, misci_v, miscf_v):
    cid = lax.axis_index("c")
    sid = lax.axis_index("s")

    @pl.when(jnp.logical_and(cid == 0, sid == 0))
    def _():
        pltpu.sync_copy(tokens_hbm, tokens_v)
        pltpu.sync_copy(scores_hbm, scores_v)
        pltpu.sync_copy(prev_hbm, prev_v)

        lane = lax.broadcasted_iota(jnp.int32, (16,), 0)
        mask4 = lane < NBEST

        sc_last = scores_v[pl.ds(T * BEAM - 16, 16)]
        neg = jnp.float32(-jnp.inf)
        cand = jnp.where(lane >= 8, sc_last, neg)
        b = jnp.zeros((16,), jnp.int32)
        for i in range(NBEST):
            m = cand
            for sh in (1, 2, 4, 8):
                rot = m.at[jnp.bitwise_and(lane + sh, 15)].get(
                    mode="promise_in_bounds")
                m = jnp.maximum(m, rot)
            j = plsc.all_reduce_ffs(cand == m)
            b = jnp.where(lane == i, j - 8, b)
            cand = jnp.where(lane == j, neg, cand)

        misci_v[...] = jnp.where(mask4, b, 0)
        sc4 = plsc.load_gather(scores_v, [NS * BEAM + b])
        miscf_v[...] = jnp.where(mask4, sc4, jnp.float32(0.0))
        pltpu.sync_copy(misci_v, ord_hbm)
        pltpu.sync_copy(miscf_v, sc_hbm)

        base4 = lane * T
        plsc.store_scatter(rows_v, [base4 + NS], jnp.zeros((16,), jnp.int32),
                           mask=mask4)

        def bt_step(t, bcur):
            idx = t * BEAM + bcur
            plsc.store_scatter(rows_v, [base4 + (t - 1)], idx, mask=mask4)
            return plsc.load_gather(prev_v, [idx])

        def bt_body(k, bcur):
            t0 = NS - k * 8
            for u in range(8):
                bcur = bt_step(t0 - u, bcur)
            return bcur

        b_tail = lax.fori_loop(0, NS // 8, bt_body, b)
        for t in range(NS - (NS // 8) * 8, 0, -1):
            b_tail = bt_step(t, b_tail)

        rotm1 = jnp.bitwise_and(lane + 15, 16 - 1)
        for i in range(NBEST):
            def blk_body(v, carry, i=i):
                off = i * T + v * 16
                ivec = rows_v[pl.ds(off, 16)]
                tokbuf_v[pl.ds(off, 16)] = plsc.load_gather(tokens_v, [ivec])
                sc = plsc.load_gather(scores_v, [ivec])
                srot = sc.at[rotm1].get(mode="promise_in_bounds")
                prev_sc = jnp.where(lane == 0, carry, srot)
                tlsbuf_v[pl.ds(off, 16)] = sc - prev_sc
                return sc[15]

            lax.fori_loop(0, T // 16, blk_body, jnp.float32(0.0))

        pltpu.sync_copy(tokbuf_v, tok_hbm)
        pltpu.sync_copy(tlsbuf_v, tls_hbm)
        pltpu.sync_copy(rows_v, rows_hbm)


_stage1 = functools.partial(
    pl.kernel,
    out_type=[
        jax.ShapeDtypeStruct((16,), jnp.int32),
        jax.ShapeDtypeStruct((16,), jnp.float32),
        jax.ShapeDtypeStruct((ROWS,), jnp.int32),
        jax.ShapeDtypeStruct((ROWS,), jnp.float32),
        jax.ShapeDtypeStruct((ROWS,), jnp.int32),
    ],
    mesh=plsc.VectorSubcoreMesh(**_MESH),
    compiler_params=pltpu.CompilerParams(needs_layout_passes=False),
    scratch_types=[
        pltpu.VMEM((T * BEAM,), jnp.int32),
        pltpu.VMEM((T * BEAM,), jnp.float32),
        pltpu.VMEM((T * BEAM,), jnp.int32),
        pltpu.VMEM((ROWS,), jnp.int32),
        pltpu.VMEM((ROWS,), jnp.int32),
        pltpu.VMEM((ROWS,), jnp.float32),
        pltpu.VMEM((16,), jnp.int32),
        pltpu.VMEM((16,), jnp.float32),
    ],
)(_stage1_body)


_CHUNK = 16
_NH = 4
_NGROUP = NBEST // _NH
_GROWS = _NH * T
_PER_W = _GROWS // 32
_NCHUNK = _PER_W // _CHUNK


_RING = 3


def _stage2_body(tw_hbm, rows_hbm, out_hbm, idx_v, buf0, buf1, buf2,
                 gsem0, gsem1, gsem2, wsem0, wsem1, wsem2):
    wid = lax.axis_index("s") * 2 + lax.axis_index("c")
    base = wid * _PER_W
    pltpu.sync_copy(rows_hbm.at[pl.ds(base, _PER_W)], idx_v)

    bufs = (buf0, buf1, buf2)
    gsems = (gsem0, gsem1, gsem2)
    wsems = (wsem0, wsem1, wsem2)

    def fire(c):
        return pltpu.async_copy(
            tw_hbm.at[idx_v.at[pl.ds(c * _CHUNK, _CHUNK)]],
            bufs[c % _RING], gsems[c % _RING])

    gcp = {}
    wcp = {}
    for c in range(min(_RING - 1, _NCHUNK)):
        gcp[c % _RING] = fire(c)
    for c in range(_NCHUNK):
        p = c % _RING
        q = (c + _RING - 1) % _RING
        if c + _RING - 1 < _NCHUNK:
            if c >= 1:
                wcp[q].wait()
            gcp[q] = fire(c + _RING - 1)
        gcp[p].wait()
        wcp[p] = pltpu.async_copy(
            bufs[p], out_hbm.at[pl.ds(base + c * _CHUNK, _CHUNK)], wsems[p])
    for c in range(max(0, _NCHUNK - _RING), _NCHUNK):
        wcp[c % _RING].wait()


_stage2 = functools.partial(
    pl.kernel,
    out_type=jax.ShapeDtypeStruct((_GROWS, SRC), jnp.float32),
    mesh=plsc.VectorSubcoreMesh(**_MESH),
    compiler_params=pltpu.CompilerParams(needs_layout_passes=False),
    scratch_types=[
        pltpu.VMEM((_PER_W,), jnp.int32),
        pltpu.VMEM((_CHUNK, SRC), jnp.float32),
        pltpu.VMEM((_CHUNK, SRC), jnp.float32),
        pltpu.VMEM((_CHUNK, SRC), jnp.float32),
        pltpu.SemaphoreType.DMA,
        pltpu.SemaphoreType.DMA,
        pltpu.SemaphoreType.DMA,
        pltpu.SemaphoreType.DMA,
        pltpu.SemaphoreType.DMA,
        pltpu.SemaphoreType.DMA,
    ],
)(_stage2_body)


_TT = 256


def _tr_body(*refs):
    xs, os = refs[:_NH], refs[_NH:]
    for x, o in zip(xs, os):
        o[...] = jnp.swapaxes(x[0], 0, 1)


def _stage3(compact):
    in_specs = [
        pl.BlockSpec((1, _TT, SRC), lambda tb, k=k: (k, tb, 0))
        for k in range(_NH)
    ]
    out_specs = [
        pl.BlockSpec((SRC, _TT), lambda tb: (0, tb)) for _ in range(_NH)
    ]
    out_shape = [
        jax.ShapeDtypeStruct((SRC, NS), jnp.float32) for _ in range(_NH)
    ]
    return pl.pallas_call(
        _tr_body,
        grid=(T // _TT,),
        in_specs=in_specs,
        out_specs=out_specs,
        out_shape=out_shape,
        compiler_params=pltpu.CompilerParams(
            vmem_limit_bytes=100 * 1024 * 1024),
    )(*([compact] * _NH))


def kernel(beam_tokens, beam_scores, token_weights, beam_prev_indices,
           num_steps):
    tokens_flat = beam_tokens.reshape(-1)
    scores_flat = beam_scores.reshape(-1)
    prev_flat = beam_prev_indices.reshape(-1)
    tw_flat = token_weights.reshape(T * BEAM, SRC)

    ord16, sc16, tokf, tlsf, rows = _stage1(tokens_flat, scores_flat,
                                            prev_flat)
    baw = []
    for g in range(_NGROUP):
        rows_g = lax.slice(rows, (g * _GROWS,), ((g + 1) * _GROWS,))
        compact = _stage2(tw_flat, rows_g)
        baw.extend(_stage3(compact.reshape(_NH, T, SRC)))

    ns_t = jnp.asarray(num_steps, jnp.int32)
    ns_f = ns_t.astype(jnp.float32)
    tok4 = tokf.reshape(NBEST, T)
    tls4 = tlsf.reshape(NBEST, T)
    outs = []
    for i in range(NBEST):
        outs.extend([
            tok4[i, :NS],
            sc16[i] / ns_f,
            tls4[i, :NS],
            baw[i],
            jnp.stack([ns_t, ord16[i]]).astype(jnp.int32),
        ])
    return tuple(outs)


# speedup vs baseline: 3.2237x; 1.0169x over previous
import functools

import jax
import jax.numpy as jnp
from jax import lax
from jax.experimental import pallas as pl
from jax.experimental.pallas import tpu as pltpu
from jax.experimental.pallas import tpu_sc as plsc

T = 2048
BEAM = 8
SRC = 2048
NBEST = 4
NS = T - 1
ROWS = NBEST * T

_MESH = dict(core_axis_name="c", subcore_axis_name="s", num_cores=2,
             num_subcores=16)

_CHUNK = 8
_PER_W = ROWS // 32
_NCHUNK = _PER_W // _CHUNK
_RING = 4


def _fused_body(tw_hbm, prev_hbm, tokens_hbm, scores_hbm,
                ord_hbm, sc_hbm, tok_hbm, tls_hbm, out_hbm,
                prev_v, sc16_v, idx_v, rowsfull_v, tokens_v, scores_v,
                tokbuf_v, tlsbuf_v, misci_v, miscf_v,
                buf0, buf1, buf2, buf3,
                gsem0, gsem1, gsem2, gsem3, wsem0, wsem1, wsem2, wsem3):
    cid = lax.axis_index("c")
    sid = lax.axis_index("s")
    wid = sid * 2 + cid
    base = wid * _PER_W
    hyp = wid // 8
    r = wid % 8
    ts = r * _PER_W

    lane = lax.broadcasted_iota(jnp.int32, (16,), 0)
    mask4 = lane < NBEST
    zeros = jnp.zeros((16,), jnp.int32)

    pltpu.sync_copy(prev_hbm, prev_v)
    pltpu.sync_copy(scores_hbm.at[pl.ds(T * BEAM - 16, 16)], sc16_v)

    sc_last = sc16_v[...]
    neg = jnp.float32(-jnp.inf)
    cand = jnp.where(lane >= 8, sc_last, neg)
    b = jnp.zeros((16,), jnp.int32)
    for i in range(NBEST):
        m = cand
        for sh in (1, 2, 4, 8):
            rot = m.at[jnp.bitwise_and(lane + sh, 15)].get(
                mode="promise_in_bounds")
            m = jnp.maximum(m, rot)
        j = plsc.all_reduce_ffs(cand == m)
        b = jnp.where(lane == i, j - 8, b)
        cand = jnp.where(lane == j, neg, cand)

    @pl.when(wid == 0)
    def _():
        misci_v[...] = jnp.where(mask4, b, 0)
        sc4 = sc_last.at[8 + b].get(mode="promise_in_bounds")
        miscf_v[...] = jnp.where(mask4, sc4, jnp.float32(0.0))
        pltpu.sync_copy(misci_v, ord_hbm)
        pltpu.sync_copy(miscf_v, sc_hbm)

    own_lane = lane == hyp
    full_lane = jnp.logical_and(own_lane, r == 0)

    plsc.store_scatter(idx_v, [zeros + (_PER_W - 1)], zeros,
                       mask=jnp.logical_and(lane == 0, r == 8 - 1))
    plsc.store_scatter(rowsfull_v, [zeros + (T - 1)], zeros, mask=full_lane)

    def bt_step(t, bcur):
        idx = t * BEAM + bcur
        in_own = jnp.logical_and(t - 1 >= ts, t - 1 < ts + _PER_W)
        plsc.store_scatter(idx_v, [zeros + jnp.bitwise_and(t - 1 - ts,
                                                           _PER_W - 1)],
                           idx, mask=jnp.logical_and(own_lane, in_own))
        plsc.store_scatter(rowsfull_v, [zeros + jnp.bitwise_and(t - 1, T - 1)],
                           idx, mask=jnp.logical_and(full_lane, t >= 1))
        return plsc.load_gather(prev_v, [idx])

    def bt_body(k, bcur):
        t0 = NS - k * 8
        for u in range(8):
            bcur = bt_step(t0 - u, bcur)
        return bcur

    lax.fori_loop(0, (NS - ts + 7) // 8, bt_body, b)

    @pl.when(r == 0)
    def _():
        pltpu.sync_copy(tokens_hbm, tokens_v)
        pltpu.sync_copy(scores_hbm, scores_v)

        rotm1 = jnp.bitwise_and(lane + 15, 16 - 1)

        def blk_body(v, carry):
            off = v * 16
            ivec = rowsfull_v[pl.ds(off, 16)]
            tokbuf_v[pl.ds(off, 16)] = plsc.load_gather(tokens_v, [ivec])
            sc = plsc.load_gather(scores_v, [ivec])
            srot = sc.at[rotm1].get(mode="promise_in_bounds")
            prev_sc = jnp.where(lane == 0, carry, srot)
            tlsbuf_v[pl.ds(off, 16)] = sc - prev_sc
            return sc[15]

        lax.fori_loop(0, T // 16, blk_body, jnp.float32(0.0))

        for hy in range(NBEST):
            @pl.when(hyp == hy)
            def _(hy=hy):
                pltpu.sync_copy(tokbuf_v, tok_hbm.at[hy])
                pltpu.sync_copy(tlsbuf_v, tls_hbm.at[hy])

    bufs = (buf0, buf1, buf2, buf3)
    gsems = (gsem0, gsem1, gsem2, gsem3)
    wsems = (wsem0, wsem1, wsem2, wsem3)

    def fire(c):
        return pltpu.async_copy(
            tw_hbm.at[idx_v.at[pl.ds(c * _CHUNK, _CHUNK)]],
            bufs[c % _RING], gsems[c % _RING])

    gcp = {}
    wcp = {}
    for c in range(min(_RING - 1, _NCHUNK)):
        gcp[c % _RING] = fire(c)
    for c in range(_NCHUNK):
        p = c % _RING
        q = (c + _RING - 1) % _RING
        if c + _RING - 1 < _NCHUNK:
            if c >= 1:
                wcp[q].wait()
            gcp[q] = fire(c + _RING - 1)
        gcp[p].wait()
        wcp[p] = pltpu.async_copy(
            bufs[p], out_hbm.at[pl.ds(base + c * _CHUNK, _CHUNK)], wsems[p])
    for c in range(max(0, _NCHUNK - _RING), _NCHUNK):
        wcp[c % _RING].wait()


_fused = functools.partial(
    pl.kernel,
    out_type=[
        jax.ShapeDtypeStruct((16,), jnp.int32),
        jax.ShapeDtypeStruct((16,), jnp.float32),
        jax.ShapeDtypeStruct((NBEST, T), jnp.int32),
        jax.ShapeDtypeStruct((NBEST, T), jnp.float32),
        jax.ShapeDtypeStruct((ROWS, SRC), jnp.float32),
    ],
    mesh=plsc.VectorSubcoreMesh(**_MESH),
    compiler_params=pltpu.CompilerParams(needs_layout_passes=False),
    scratch_types=[
        pltpu.VMEM((T * BEAM,), jnp.int32),
        pltpu.VMEM((16,), jnp.float32),
        pltpu.VMEM((_PER_W,), jnp.int32),
        pltpu.VMEM((T,), jnp.int32),
        pltpu.VMEM((T * BEAM,), jnp.int32),
        pltpu.VMEM((T * BEAM,), jnp.float32),
        pltpu.VMEM((T,), jnp.int32),
        pltpu.VMEM((T,), jnp.float32),
        pltpu.VMEM((16,), jnp.int32),
        pltpu.VMEM((16,), jnp.float32),
        pltpu.VMEM((_CHUNK, SRC), jnp.float32),
        pltpu.VMEM((_CHUNK, SRC), jnp.float32),
        pltpu.VMEM((_CHUNK, SRC), jnp.float32),
        pltpu.VMEM((_CHUNK, SRC), jnp.float32),
        pltpu.SemaphoreType.DMA,
        pltpu.SemaphoreType.DMA,
        pltpu.SemaphoreType.DMA,
        pltpu.SemaphoreType.DMA,
        pltpu.SemaphoreType.DMA,
        pltpu.SemaphoreType.DMA,
        pltpu.SemaphoreType.DMA,
        pltpu.SemaphoreType.DMA,
    ],
)(_fused_body)


_TT = 256


def _tr_body(*refs):
    xs, os = refs[:NBEST], refs[NBEST:]
    for x, o in zip(xs, os):
        o[...] = jnp.swapaxes(x[0], 0, 1)


def _stage3(compact):
    in_specs = [
        pl.BlockSpec((1, _TT, SRC), lambda tb, k=k: (k, tb, 0))
        for k in range(NBEST)
    ]
    out_specs = [
        pl.BlockSpec((SRC, _TT), lambda tb: (0, tb)) for _ in range(NBEST)
    ]
    out_shape = [
        jax.ShapeDtypeStruct((SRC, NS), jnp.float32) for _ in range(NBEST)
    ]
    return pl.pallas_call(
        _tr_body,
        grid=(T // _TT,),
        in_specs=in_specs,
        out_specs=out_specs,
        out_shape=out_shape,
        compiler_params=pltpu.CompilerParams(
            vmem_limit_bytes=100 * 1024 * 1024),
    )(*([compact] * NBEST))


def kernel(beam_tokens, beam_scores, token_weights, beam_prev_indices,
           num_steps):
    tokens_flat = beam_tokens.reshape(-1)
    scores_flat = beam_scores.reshape(-1)
    prev_flat = beam_prev_indices.reshape(-1)
    tw_flat = token_weights.reshape(T * BEAM, SRC)
    ord16, sc16, tok4, tls4, compact = _fused(tw_flat, prev_flat,
                                              tokens_flat, scores_flat)
    baw = _stage3(compact.reshape(NBEST, T, SRC))

    ns_t = jnp.asarray(num_steps, jnp.int32)
    ns_f = ns_t.astype(jnp.float32)
    outs = []
    for i in range(NBEST):
        outs.extend([
            tok4[i, :NS],
            sc16[i] / ns_f,
            tls4[i, :NS],
            baw[i],
            jnp.stack([ns_t, ord16[i]]).astype(jnp.int32),
        ])
    return tuple(outs)


# speedup vs baseline: 3.2521x; 1.0088x over previous
import functools

import jax
import jax.numpy as jnp
from jax import lax
from jax.experimental import pallas as pl
from jax.experimental.pallas import tpu as pltpu
from jax.experimental.pallas import tpu_sc as plsc

T = 2048
BEAM = 8
SRC = 2048
NBEST = 4
NS = T - 1
ROWS = NBEST * T

_MESH = dict(core_axis_name="c", subcore_axis_name="s", num_cores=2,
             num_subcores=16)

_CHUNK = 8
_PER_W = ROWS // 32
_NCHUNK = _PER_W // _CHUNK
_RING = 4


def _fused_body(tw_hbm, prev_hbm, tokens_hbm, scores_hbm,
                ord_hbm, sc_hbm, tok_hbm, tls_hbm, out_hbm,
                prev_v, sc16_v, idx_v, rowsfull_v, tokens_v, scores_v,
                tokbuf_v, tlsbuf_v, misci_v, miscf_v,
                buf0, buf1, buf2, buf3,
                gsem0, gsem1, gsem2, gsem3, wsem0, wsem1, wsem2, wsem3):
    cid = lax.axis_index("c")
    sid = lax.axis_index("s")
    wid = cid * 16 + sid
    base = wid * _PER_W
    hyp = wid // 8
    r = wid % 8
    ts = r * _PER_W

    lane = lax.broadcasted_iota(jnp.int32, (16,), 0)
    mask4 = lane < NBEST
    zeros = jnp.zeros((16,), jnp.int32)

    pltpu.sync_copy(prev_hbm, prev_v)
    pltpu.sync_copy(scores_hbm.at[pl.ds(T * BEAM - 16, 16)], sc16_v)

    sc_last = sc16_v[...]
    neg = jnp.float32(-jnp.inf)
    cand = jnp.where(lane >= 8, sc_last, neg)
    b = jnp.zeros((16,), jnp.int32)
    for i in range(NBEST):
        m = cand
        for sh in (1, 2, 4, 8):
            rot = m.at[jnp.bitwise_and(lane + sh, 15)].get(
                mode="promise_in_bounds")
            m = jnp.maximum(m, rot)
        j = plsc.all_reduce_ffs(cand == m)
        b = jnp.where(lane == i, j - 8, b)
        cand = jnp.where(lane == j, neg, cand)

    @pl.when(wid == 0)
    def _():
        misci_v[...] = jnp.where(mask4, b, 0)
        sc4 = sc_last.at[8 + b].get(mode="promise_in_bounds")
        miscf_v[...] = jnp.where(mask4, sc4, jnp.float32(0.0))
        pltpu.sync_copy(misci_v, ord_hbm)
        pltpu.sync_copy(miscf_v, sc_hbm)

    own_lane = lane == hyp
    full_lane = jnp.logical_and(own_lane, r == 0)

    plsc.store_scatter(idx_v, [zeros + (_PER_W - 1)], zeros,
                       mask=jnp.logical_and(lane == 0, r == 8 - 1))
    plsc.store_scatter(rowsfull_v, [zeros + (T - 1)], zeros, mask=full_lane)

    def bt_step(t, bcur):
        idx = t * BEAM + bcur
        in_own = jnp.logical_and(t - 1 >= ts, t - 1 < ts + _PER_W)
        plsc.store_scatter(idx_v, [zeros + jnp.bitwise_and(t - 1 - ts,
                                                           _PER_W - 1)],
                           idx, mask=jnp.logical_and(own_lane, in_own))
        plsc.store_scatter(rowsfull_v, [zeros + jnp.bitwise_and(t - 1, T - 1)],
                           idx, mask=jnp.logical_and(full_lane, t >= 1))
        return plsc.load_gather(prev_v, [idx])

    def bt_body(k, bcur):
        t0 = NS - k * 8
        for u in range(8):
            bcur = bt_step(t0 - u, bcur)
        return bcur

    @pl.when(r == 0)
    def _():
        pltpu.sync_copy(tokens_hbm, tokens_v)
        pltpu.sync_copy(scores_hbm, scores_v)

    lax.fori_loop(0, (NS - ts + 7) // 8, bt_body, b)

    def post_pass():
        rotm1 = jnp.bitwise_and(lane + 15, 16 - 1)

        def blk_body(v, carry):
            off = v * 16
            ivec = rowsfull_v[pl.ds(off, 16)]
            tokbuf_v[pl.ds(off, 16)] = plsc.load_gather(tokens_v, [ivec])
            sc = plsc.load_gather(scores_v, [ivec])
            srot = sc.at[rotm1].get(mode="promise_in_bounds")
            prev_sc = jnp.where(lane == 0, carry, srot)
            tlsbuf_v[pl.ds(off, 16)] = sc - prev_sc
            return sc[15]

        lax.fori_loop(0, T // 16, blk_body, jnp.float32(0.0))

        for hy in range(NBEST):
            @pl.when(hyp == hy)
            def _(hy=hy):
                pltpu.sync_copy(tokbuf_v, tok_hbm.at[hy])
                pltpu.sync_copy(tlsbuf_v, tls_hbm.at[hy])

    bufs = (buf0, buf1, buf2, buf3)
    gsems = (gsem0, gsem1, gsem2, gsem3)
    wsems = (wsem0, wsem1, wsem2, wsem3)

    def fire(c):
        return pltpu.async_copy(
            tw_hbm.at[idx_v.at[pl.ds(c * _CHUNK, _CHUNK)]],
            bufs[c % _RING], gsems[c % _RING])

    gcp = {}
    wcp = {}
    for c in range(min(_RING - 1, _NCHUNK)):
        gcp[c % _RING] = fire(c)

    @pl.when(r == 0)
    def _():
        post_pass()

    for c in range(_NCHUNK):
        p = c % _RING
        q = (c + _RING - 1) % _RING
        if c + _RING - 1 < _NCHUNK:
            if c >= 1:
                wcp[q].wait()
            gcp[q] = fire(c + _RING - 1)
        gcp[p].wait()
        wcp[p] = pltpu.async_copy(
            bufs[p], out_hbm.at[pl.ds(base + c * _CHUNK, _CHUNK)], wsems[p])
    for c in range(max(0, _NCHUNK - _RING), _NCHUNK):
        wcp[c % _RING].wait()


_fused = functools.partial(
    pl.kernel,
    out_type=[
        jax.ShapeDtypeStruct((16,), jnp.int32),
        jax.ShapeDtypeStruct((16,), jnp.float32),
        jax.ShapeDtypeStruct((NBEST, T), jnp.int32),
        jax.ShapeDtypeStruct((NBEST, T), jnp.float32),
        jax.ShapeDtypeStruct((ROWS, SRC), jnp.float32),
    ],
    mesh=plsc.VectorSubcoreMesh(**_MESH),
    compiler_params=pltpu.CompilerParams(needs_layout_passes=False),
    scratch_types=[
        pltpu.VMEM((T * BEAM,), jnp.int32),
        pltpu.VMEM((16,), jnp.float32),
        pltpu.VMEM((_PER_W,), jnp.int32),
        pltpu.VMEM((T,), jnp.int32),
        pltpu.VMEM((T * BEAM,), jnp.int32),
        pltpu.VMEM((T * BEAM,), jnp.float32),
        pltpu.VMEM((T,), jnp.int32),
        pltpu.VMEM((T,), jnp.float32),
        pltpu.VMEM((16,), jnp.int32),
        pltpu.VMEM((16,), jnp.float32),
        pltpu.VMEM((_CHUNK, SRC), jnp.float32),
        pltpu.VMEM((_CHUNK, SRC), jnp.float32),
        pltpu.VMEM((_CHUNK, SRC), jnp.float32),
        pltpu.VMEM((_CHUNK, SRC), jnp.float32),
        pltpu.SemaphoreType.DMA,
        pltpu.SemaphoreType.DMA,
        pltpu.SemaphoreType.DMA,
        pltpu.SemaphoreType.DMA,
        pltpu.SemaphoreType.DMA,
        pltpu.SemaphoreType.DMA,
        pltpu.SemaphoreType.DMA,
        pltpu.SemaphoreType.DMA,
    ],
)(_fused_body)


_TT = 256


def _tr_body(*refs):
    xs, os = refs[:NBEST], refs[NBEST:]
    for x, o in zip(xs, os):
        o[...] = jnp.swapaxes(x[0], 0, 1)


def _stage3(compact):
    in_specs = [
        pl.BlockSpec((1, _TT, SRC), lambda tb, k=k: (k, tb, 0))
        for k in range(NBEST)
    ]
    out_specs = [
        pl.BlockSpec((SRC, _TT), lambda tb: (0, tb)) for _ in range(NBEST)
    ]
    out_shape = [
        jax.ShapeDtypeStruct((SRC, NS), jnp.float32) for _ in range(NBEST)
    ]
    return pl.pallas_call(
        _tr_body,
        grid=(T // _TT,),
        in_specs=in_specs,
        out_specs=out_specs,
        out_shape=out_shape,
        compiler_params=pltpu.CompilerParams(
            vmem_limit_bytes=100 * 1024 * 1024),
    )(*([compact] * NBEST))


def kernel(beam_tokens, beam_scores, token_weights, beam_prev_indices,
           num_steps):
    tokens_flat = beam_tokens.reshape(-1)
    scores_flat = beam_scores.reshape(-1)
    prev_flat = beam_prev_indices.reshape(-1)
    tw_flat = token_weights.reshape(T * BEAM, SRC)
    ord16, sc16, tok4, tls4, compact = _fused(tw_flat, prev_flat,
                                              tokens_flat, scores_flat)
    baw = _stage3(compact.reshape(NBEST, T, SRC))

    ns_t = jnp.asarray(num_steps, jnp.int32)
    ns_f = ns_t.astype(jnp.float32)
    outs = []
    for i in range(NBEST):
        outs.extend([
            tok4[i, :NS],
            sc16[i] / ns_f,
            tls4[i, :NS],
            baw[i],
            jnp.stack([ns_t, ord16[i]]).astype(jnp.int32),
        ])
    return tuple(outs)


# speedup vs baseline: 3.2723x; 1.0062x over previous
import functools

import jax
import jax.numpy as jnp
from jax import lax
from jax.experimental import pallas as pl
from jax.experimental.pallas import tpu as pltpu
from jax.experimental.pallas import tpu_sc as plsc

T = 2048
BEAM = 8
SRC = 2048
NBEST = 4
NS = T - 1
ROWS = NBEST * T

_MESH = dict(core_axis_name="c", subcore_axis_name="s", num_cores=2,
             num_subcores=16)

_CHUNK = 16
_PER_W = ROWS // 32
_NCHUNK = _PER_W // _CHUNK
_RING = 2


def _fused_body(tw_hbm, prev_hbm, tokens_hbm, scores_hbm,
                ord_hbm, sc_hbm, tok_hbm, tls_hbm, out_hbm,
                prev_v, sc16_v, idx_v, rowsfull_v, tokens_v, scores_v,
                tokbuf_v, tlsbuf_v, misci_v, miscf_v,
                buf0, buf1,
                gsem0, gsem1, wsem0, wsem1):
    cid = lax.axis_index("c")
    sid = lax.axis_index("s")
    wid = cid * 16 + sid
    base = wid * _PER_W
    hyp = wid // 8
    r = wid % 8
    ts = r * _PER_W

    lane = lax.broadcasted_iota(jnp.int32, (16,), 0)
    mask4 = lane < NBEST
    zeros = jnp.zeros((16,), jnp.int32)

    pltpu.sync_copy(prev_hbm, prev_v)
    pltpu.sync_copy(scores_hbm.at[pl.ds(T * BEAM - 16, 16)], sc16_v)

    sc_last = sc16_v[...]
    neg = jnp.float32(-jnp.inf)
    cand = jnp.where(lane >= 8, sc_last, neg)
    b = jnp.zeros((16,), jnp.int32)
    for i in range(NBEST):
        m = cand
        for sh in (1, 2, 4, 8):
            rot = m.at[jnp.bitwise_and(lane + sh, 15)].get(
                mode="promise_in_bounds")
            m = jnp.maximum(m, rot)
        j = plsc.all_reduce_ffs(cand == m)
        b = jnp.where(lane == i, j - 8, b)
        cand = jnp.where(lane == j, neg, cand)

    @pl.when(wid == 0)
    def _():
        misci_v[...] = jnp.where(mask4, b, 0)
        sc4 = sc_last.at[8 + b].get(mode="promise_in_bounds")
        miscf_v[...] = jnp.where(mask4, sc4, jnp.float32(0.0))
        pltpu.sync_copy(misci_v, ord_hbm)
        pltpu.sync_copy(miscf_v, sc_hbm)

    own_lane = lane == hyp
    full_lane = jnp.logical_and(own_lane, r == 0)

    plsc.store_scatter(idx_v, [zeros + (_PER_W - 1)], zeros,
                       mask=jnp.logical_and(lane == 0, r == 8 - 1))
    plsc.store_scatter(rowsfull_v, [zeros + (T - 1)], zeros, mask=full_lane)

    def bt_step(t, bcur):
        idx = t * BEAM + bcur
        in_own = jnp.logical_and(t - 1 >= ts, t - 1 < ts + _PER_W)
        plsc.store_scatter(idx_v, [zeros + jnp.bitwise_and(t - 1 - ts,
                                                           _PER_W - 1)],
                           idx, mask=jnp.logical_and(own_lane, in_own))
        plsc.store_scatter(rowsfull_v, [zeros + jnp.bitwise_and(t - 1, T - 1)],
                           idx, mask=jnp.logical_and(full_lane, t >= 1))
        return plsc.load_gather(prev_v, [idx])

    def bt_body(k, bcur):
        t0 = NS - k * 8
        for u in range(8):
            bcur = bt_step(t0 - u, bcur)
        return bcur

    @pl.when(r == 0)
    def _():
        pltpu.sync_copy(tokens_hbm, tokens_v)
        pltpu.sync_copy(scores_hbm, scores_v)

    lax.fori_loop(0, (NS - ts + 7) // 8, bt_body, b)

    def post_pass():
        rotm1 = jnp.bitwise_and(lane + 15, 16 - 1)

        def blk_body(v, carry):
            off = v * 16
            ivec = rowsfull_v[pl.ds(off, 16)]
            tokbuf_v[pl.ds(off, 16)] = plsc.load_gather(tokens_v, [ivec])
            sc = plsc.load_gather(scores_v, [ivec])
            srot = sc.at[rotm1].get(mode="promise_in_bounds")
            prev_sc = jnp.where(lane == 0, carry, srot)
            tlsbuf_v[pl.ds(off, 16)] = sc - prev_sc
            return sc[15]

        lax.fori_loop(0, T // 16, blk_body, jnp.float32(0.0))

        for hy in range(NBEST):
            @pl.when(hyp == hy)
            def _(hy=hy):
                pltpu.sync_copy(tokbuf_v, tok_hbm.at[hy])
                pltpu.sync_copy(tlsbuf_v, tls_hbm.at[hy])

    bufs = (buf0, buf1)
    gsems = (gsem0, gsem1)
    wsems = (wsem0, wsem1)

    def fire(c):
        return pltpu.async_copy(
            tw_hbm.at[idx_v.at[pl.ds(c * _CHUNK, _CHUNK)]],
            bufs[c % _RING], gsems[c % _RING])

    gcp = {}
    wcp = {}
    for c in range(min(_RING - 1, _NCHUNK)):
        gcp[c % _RING] = fire(c)

    @pl.when(r == 0)
    def _():
        post_pass()

    for c in range(_NCHUNK):
        p = c % _RING
        q = (c + _RING - 1) % _RING
        if c + _RING - 1 < _NCHUNK:
            if c >= 1:
                wcp[q].wait()
            gcp[q] = fire(c + _RING - 1)
        gcp[p].wait()
        wcp[p] = pltpu.async_copy(
            bufs[p], out_hbm.at[pl.ds(base + c * _CHUNK, _CHUNK)], wsems[p])
    for c in range(max(0, _NCHUNK - _RING), _NCHUNK):
        wcp[c % _RING].wait()


_fused = functools.partial(
    pl.kernel,
    out_type=[
        jax.ShapeDtypeStruct((16,), jnp.int32),
        jax.ShapeDtypeStruct((16,), jnp.float32),
        jax.ShapeDtypeStruct((NBEST, T), jnp.int32),
        jax.ShapeDtypeStruct((NBEST, T), jnp.float32),
        jax.ShapeDtypeStruct((ROWS, SRC), jnp.float32),
    ],
    mesh=plsc.VectorSubcoreMesh(**_MESH),
    compiler_params=pltpu.CompilerParams(needs_layout_passes=False),
    scratch_types=[
        pltpu.VMEM((T * BEAM,), jnp.int32),
        pltpu.VMEM((16,), jnp.float32),
        pltpu.VMEM((_PER_W,), jnp.int32),
        pltpu.VMEM((T,), jnp.int32),
        pltpu.VMEM((T * BEAM,), jnp.int32),
        pltpu.VMEM((T * BEAM,), jnp.float32),
        pltpu.VMEM((T,), jnp.int32),
        pltpu.VMEM((T,), jnp.float32),
        pltpu.VMEM((16,), jnp.int32),
        pltpu.VMEM((16,), jnp.float32),
        pltpu.VMEM((_CHUNK, SRC), jnp.float32),
        pltpu.VMEM((_CHUNK, SRC), jnp.float32),
        pltpu.SemaphoreType.DMA,
        pltpu.SemaphoreType.DMA,
        pltpu.SemaphoreType.DMA,
        pltpu.SemaphoreType.DMA,
    ],
)(_fused_body)


_TT = 256


def _tr_body(*refs):
    xs, os = refs[:NBEST], refs[NBEST:]
    for x, o in zip(xs, os):
        o[...] = jnp.swapaxes(x[0], 0, 1)


def _stage3(compact):
    in_specs = [
        pl.BlockSpec((1, _TT, SRC), lambda tb, k=k: (k, tb, 0))
        for k in range(NBEST)
    ]
    out_specs = [
        pl.BlockSpec((SRC, _TT), lambda tb: (0, tb)) for _ in range(NBEST)
    ]
    out_shape = [
        jax.ShapeDtypeStruct((SRC, NS), jnp.float32) for _ in range(NBEST)
    ]
    return pl.pallas_call(
        _tr_body,
        grid=(T // _TT,),
        in_specs=in_specs,
        out_specs=out_specs,
        out_shape=out_shape,
        compiler_params=pltpu.CompilerParams(
            vmem_limit_bytes=100 * 1024 * 1024),
    )(*([compact] * NBEST))


def kernel(beam_tokens, beam_scores, token_weights, beam_prev_indices,
           num_steps):
    tokens_flat = beam_tokens.reshape(-1)
    scores_flat = beam_scores.reshape(-1)
    prev_flat = beam_prev_indices.reshape(-1)
    tw_flat = token_weights.reshape(T * BEAM, SRC)
    ord16, sc16, tok4, tls4, compact = _fused(tw_flat, prev_flat,
                                              tokens_flat, scores_flat)
    baw = _stage3(compact.reshape(NBEST, T, SRC))

    ns_t = jnp.asarray(num_steps, jnp.int32)
    ns_f = ns_t.astype(jnp.float32)
    outs = []
    for i in range(NBEST):
        outs.extend([
            tok4[i, :NS],
            sc16[i] / ns_f,
            tls4[i, :NS],
            baw[i],
            jnp.stack([ns_t, ord16[i]]).astype(jnp.int32),
        ])
    return tuple(outs)
